# baseline (speedup 1.0000x reference)
import jax
import jax.numpy as jnp
from jax.experimental import pallas as pl
from jax.experimental.pallas import tpu as pltpu
from jax.experimental.pallas import tpu_sc as plsc

_B, _S, _D, _E = 4, 2048, 1024, 3
_S_BLK = 1024
_S_BLKS = _S // _S_BLK

_CH = 256
_CHUNKS = _S // _CH
_TOTAL = _B * _CHUNKS
_NBUF = 8
_LAG = 4


_SCH = 256
_SCHUNKS = _S // _SCH
_STOTAL = _B * _SCHUNKS
_SNBUF = 6


def _stats_body(v0_ref, a0_ref, w_ref, bias_ref, logits_ref,
                vbuf_ref, abuf_ref, acc_ref, vsem, asem):
    def read_start(t):
        b, j = divmod(t, _SCHUNKS)
        rows = pl.ds(j * _SCH, _SCH)
        slot = t % _SNBUF
        pltpu.make_async_copy(
            v0_ref.at[b, rows, :], vbuf_ref.at[slot], vsem.at[slot]
        ).start()
        pltpu.make_async_copy(
            a0_ref.at[b, rows, :], abuf_ref.at[slot], asem.at[slot]
        ).start()

    def read_wait(t):
        b, j = divmod(t, _SCHUNKS)
        rows = pl.ds(j * _SCH, _SCH)
        slot = t % _SNBUF
        pltpu.make_async_copy(
            v0_ref.at[b, rows, :], vbuf_ref.at[slot], vsem.at[slot]
        ).wait()
        pltpu.make_async_copy(
            a0_ref.at[b, rows, :], abuf_ref.at[slot], asem.at[slot]
        ).wait()

    for t in range(min(_SNBUF, _STOTAL)):
        read_start(t)

    def chunk_sums(buf, slot):
        x0 = buf[slot, 0:8, :]
        s, q = x0, x0 * x0
        for k in range(1, _SCH // 8):
            x = buf[slot, 8 * k:8 * (k + 1), :]
            s = s + x
            q = q + x * x
        return s, q

    for t in range(_STOTAL):
        b, j = divmod(t, _SCHUNKS)
        slot = t % _SNBUF
        read_wait(t)
        sv, qv = chunk_sums(vbuf_ref, slot)
        sa, qa = chunk_sums(abuf_ref, slot)
        if j == 0:
            acc_ref[0:8, :] = sv
            acc_ref[8:16, :] = qv
            acc_ref[16:24, :] = sa
            acc_ref[24:32, :] = qa
        else:
            acc_ref[0:8, :] += sv
            acc_ref[8:16, :] += qv
            acc_ref[16:24, :] += sa
            acc_ref[24:32, :] += qa
        if t + _SNBUF < _STOTAL:
            read_start(t + _SNBUF)
        if j == _SCHUNKS - 1:
            inv_s = 1.0 / _S
            inv_n1 = 1.0 / (_S - 1)
            s_v = jnp.sum(acc_ref[0:8, :], axis=0, keepdims=True)
            q_v = jnp.sum(acc_ref[8:16, :], axis=0, keepdims=True)
            s_a = jnp.sum(acc_ref[16:24, :], axis=0, keepdims=True)
            q_a = jnp.sum(acc_ref[24:32, :], axis=0, keepdims=True)
            mean_v = s_v * inv_s
            var_v = (q_v - _S * mean_v * mean_v) * inv_n1
            mean_a = s_a * inv_s
            var_a = (q_a - _S * mean_a * mean_a) * inv_n1
            feats = jnp.concatenate(
                [mean_v, jnp.sqrt(var_v), mean_a, jnp.sqrt(var_a)], axis=1
            )
            logits = jnp.sum(w_ref[...] * feats, axis=1) + bias_ref[0]
            logits_ref[b, :] = logits


def _argmax3(lg_ref, b):
    l0 = lg_ref[3 * b]
    l1 = lg_ref[3 * b + 1]
    l2 = lg_ref[3 * b + 2]
    i01 = jnp.where(l1 > l0, 1, 0)
    m01 = jnp.maximum(l0, l1)
    return jnp.where(l2 > m01, 2, i01)


def _copy_body(lg_ref, v_ref, a_ref, av_ref, out_ref, buf_ref, rsem, wsem):
    es = [_argmax3(lg_ref, b) for b in range(_B)]

    def chunk(t):
        b, j = divmod(t, _CHUNKS)
        return b, pl.ds(j * _CH, _CH), t % _NBUF

    def read_start(t):
        b, rows, slot = chunk(t)
        for e, src in ((0, v_ref), (1, a_ref), (2, av_ref)):
            @pl.when(es[b] == e)
            def _(src=src):
                pltpu.make_async_copy(
                    src.at[b, rows, :], buf_ref.at[slot], rsem.at[slot]
                ).start()

    def read_wait(t):
        b, rows, slot = chunk(t)
        pltpu.make_async_copy(
            v_ref.at[b, rows, :], buf_ref.at[slot], rsem.at[slot]
        ).wait()

    def write_copy(t):
        b, rows, slot = chunk(t)
        return pltpu.make_async_copy(
            buf_ref.at[slot], out_ref.at[b, rows, :], wsem.at[slot]
        )

    for t in range(_TOTAL + _LAG):
        if t < _TOTAL:
            if t >= _NBUF:
                write_copy(t - _NBUF).wait()
            read_start(t)
        if t >= _LAG:
            read_wait(t - _LAG)
            write_copy(t - _LAG).start()
    for t in range(_TOTAL - _NBUF, _TOTAL):
        write_copy(t).wait()


_SC_NBUF = 4
_SC_LAG = 2
_SC_CH = 256
_SC_CHUNKS = _S // _SC_CH
_SC_BPC = _B // 2
_SC_TOT = _SC_BPC * _SC_CHUNKS


def _sc_copy_body(lg_ref, v_ref, a_ref, av_ref, o_ref,
                  lg_smem, buf_ref, lsem, rsem, wsem):
    core = jax.lax.axis_index("core")
    pltpu.make_async_copy(lg_ref, lg_smem, lsem).start()
    pltpu.make_async_copy(lg_ref, lg_smem, lsem).wait()

    def eb(b):
        l0 = lg_smem[3 * b]
        l1 = lg_smem[3 * b + 1]
        l2 = lg_smem[3 * b + 2]
        return jnp.where(l2 > jnp.maximum(l0, l1), 2,
                         jnp.where(l1 > l0, 1, 0))

    es = [eb(core * _SC_BPC + i) for i in range(_SC_BPC)]

    def chunk(t):
        i, j = divmod(t, _SC_CHUNKS)
        b = core * _SC_BPC + i
        return i, b, pl.ds(j * _SC_CH, _SC_CH), t % _SC_NBUF

    def read_start(t):
        i, b, rows, slot = chunk(t)
        for k, src in ((0, v_ref), (1, a_ref), (2, av_ref)):
            @pl.when(es[i] == k)
            def _(src=src):
                pltpu.make_async_copy(
                    src.at[b, rows, :], buf_ref.at[slot], rsem.at[slot]
                ).start()

    def read_wait(t):
        i, b, rows, slot = chunk(t)
        pltpu.make_async_copy(
            v_ref.at[b, rows, :], buf_ref.at[slot], rsem.at[slot]
        ).wait()

    def write_copy(t):
        i, b, rows, slot = chunk(t)
        return pltpu.make_async_copy(
            buf_ref.at[slot], o_ref.at[b, rows, :], wsem.at[slot]
        )

    for t in range(_SC_TOT + _SC_LAG):
        if t < _SC_TOT:
            if t >= _SC_NBUF:
                write_copy(t - _SC_NBUF).wait()
            read_start(t)
        if t >= _SC_LAG:
            read_wait(t - _SC_LAG)
            write_copy(t - _SC_LAG).start()
    for t in range(_SC_TOT - _SC_NBUF, _SC_TOT):
        write_copy(t).wait()


def _sc_copy(logits, v, a, av):
    lg16 = jnp.zeros((16,), jnp.float32).at[: _B * _E].set(logits.reshape(-1))
    f = pl.kernel(
        _sc_copy_body,
        out_type=jax.ShapeDtypeStruct((_B, _S, _D), jnp.float32),
        mesh=plsc.ScalarSubcoreMesh(axis_name="core", num_cores=2),
        scratch_types=[
            pltpu.SMEM((16,), jnp.float32),
            pltpu.VMEM_SHARED((_SC_NBUF, _SC_CH, _D), jnp.float32),
            pltpu.SemaphoreType.DMA,
            pltpu.SemaphoreType.DMA((_SC_NBUF,)),
            pltpu.SemaphoreType.DMA((_SC_NBUF,)),
        ],
    )
    return f(lg16, v, a, av)


def kernel(v0, a0, v, a, av, W, b):
    logits = pl.pallas_call(
        _stats_body,
        grid=(1,),
        in_specs=[
            pl.BlockSpec(memory_space=pl.ANY),
            pl.BlockSpec(memory_space=pl.ANY),
            pl.BlockSpec((_E, 4 * _D), lambda i: (0, 0)),
            pl.BlockSpec((1, _E), lambda i: (0, 0)),
        ],
        out_specs=pl.BlockSpec((_B, _E), lambda i: (0, 0)),
        out_shape=jax.ShapeDtypeStruct((_B, _E), jnp.float32),
        scratch_shapes=[
            pltpu.VMEM((_SNBUF, _SCH, _D), jnp.float32),
            pltpu.VMEM((_SNBUF, _SCH, _D), jnp.float32),
            pltpu.VMEM((32, _D), jnp.float32),
            pltpu.SemaphoreType.DMA((_SNBUF,)),
            pltpu.SemaphoreType.DMA((_SNBUF,)),
        ],
    )(v0, a0, W, b.reshape(1, _E))

    combined = _sc_copy(logits, v, a, av)

    return combined, logits


# speedup vs baseline: 1.0160x; 1.0160x over previous
import functools

import jax
import jax.numpy as jnp
from jax.experimental import pallas as pl
from jax.experimental.pallas import tpu as pltpu
from jax.experimental.pallas import tpu_sc as plsc

_B, _S, _D, _E = 4, 2048, 1024, 3

_CH = 256
_CHUNKS = _S // _CH
_NBUF = 8
_LAG = 4

_SCH = 256
_SCHUNKS = _S // _SCH
_SNBUF = 6


def _stats_body(v0_ref, a0_ref, w_ref, bias_ref, logits_ref,
                vbuf_ref, abuf_ref, acc_ref, vsem, asem, *, batches):
    total = len(batches) * _SCHUNKS

    def read_start(t):
        i, j = divmod(t, _SCHUNKS)
        b = batches[i]
        rows = pl.ds(j * _SCH, _SCH)
        slot = t % _SNBUF
        pltpu.make_async_copy(
            v0_ref.at[b, rows, :], vbuf_ref.at[slot], vsem.at[slot]
        ).start()
        pltpu.make_async_copy(
            a0_ref.at[b, rows, :], abuf_ref.at[slot], asem.at[slot]
        ).start()

    def read_wait(t):
        i, j = divmod(t, _SCHUNKS)
        b = batches[i]
        rows = pl.ds(j * _SCH, _SCH)
        slot = t % _SNBUF
        pltpu.make_async_copy(
            v0_ref.at[b, rows, :], vbuf_ref.at[slot], vsem.at[slot]
        ).wait()
        pltpu.make_async_copy(
            a0_ref.at[b, rows, :], abuf_ref.at[slot], asem.at[slot]
        ).wait()

    for t in range(min(_SNBUF, total)):
        read_start(t)

    def chunk_sums(buf, slot):
        gs, gq = [], []
        for g in range(4):
            base = g * (_SCH // 4)
            x0 = buf[slot, base:base + 8, :]
            s, q = x0, x0 * x0
            for k in range(1, _SCH // 32):
                x = buf[slot, base + 8 * k:base + 8 * (k + 1), :]
                s = s + x
                q = q + x * x
            gs.append(s)
            gq.append(q)
        return (gs[0] + gs[1]) + (gs[2] + gs[3]), (gq[0] + gq[1]) + (gq[2] + gq[3])

    for t in range(total):
        i, j = divmod(t, _SCHUNKS)
        slot = t % _SNBUF
        read_wait(t)
        sv, qv = chunk_sums(vbuf_ref, slot)
        sa, qa = chunk_sums(abuf_ref, slot)
        if j == 0:
            acc_ref[0:8, :] = sv
            acc_ref[8:16, :] = qv
            acc_ref[16:24, :] = sa
            acc_ref[24:32, :] = qa
        else:
            acc_ref[0:8, :] += sv
            acc_ref[8:16, :] += qv
            acc_ref[16:24, :] += sa
            acc_ref[24:32, :] += qa
        if t + _SNBUF < total:
            read_start(t + _SNBUF)
        if j == _SCHUNKS - 1:
            inv_s = 1.0 / _S
            inv_n1 = 1.0 / (_S - 1)
            s_v = jnp.sum(acc_ref[0:8, :], axis=0, keepdims=True)
            q_v = jnp.sum(acc_ref[8:16, :], axis=0, keepdims=True)
            s_a = jnp.sum(acc_ref[16:24, :], axis=0, keepdims=True)
            q_a = jnp.sum(acc_ref[24:32, :], axis=0, keepdims=True)
            mean_v = s_v * inv_s
            var_v = (q_v - _S * mean_v * mean_v) * inv_n1
            mean_a = s_a * inv_s
            var_a = (q_a - _S * mean_a * mean_a) * inv_n1
            feats = jnp.concatenate(
                [mean_v, jnp.sqrt(var_v), mean_a, jnp.sqrt(var_a)], axis=1
            )
            w_bf = w_ref[...].astype(jnp.bfloat16).astype(jnp.float32)
            f_bf = feats.astype(jnp.bfloat16).astype(jnp.float32)
            logits = jnp.sum(w_bf * f_bf, axis=1) + bias_ref[0]
            logits_ref[i, :] = logits


def _stats(v0, a0, W, bias, batches):
    return pl.pallas_call(
        functools.partial(_stats_body, batches=batches),
        grid=(1,),
        in_specs=[
            pl.BlockSpec(memory_space=pl.ANY),
            pl.BlockSpec(memory_space=pl.ANY),
            pl.BlockSpec((_E, 4 * _D), lambda i: (0, 0)),
            pl.BlockSpec((1, _E), lambda i: (0, 0)),
        ],
        out_specs=pl.BlockSpec((len(batches), _E), lambda i: (0, 0)),
        out_shape=jax.ShapeDtypeStruct((len(batches), _E), jnp.float32),
        scratch_shapes=[
            pltpu.VMEM((_SNBUF, _SCH, _D), jnp.float32),
            pltpu.VMEM((_SNBUF, _SCH, _D), jnp.float32),
            pltpu.VMEM((32, _D), jnp.float32),
            pltpu.SemaphoreType.DMA((_SNBUF,)),
            pltpu.SemaphoreType.DMA((_SNBUF,)),
        ],
    )(v0, a0, W, bias)


def _argmax3(lg_ref, b):
    l0 = lg_ref[3 * b]
    l1 = lg_ref[3 * b + 1]
    l2 = lg_ref[3 * b + 2]
    i01 = jnp.where(l1 > l0, 1, 0)
    m01 = jnp.maximum(l0, l1)
    return jnp.where(l2 > m01, 2, i01)


_TC_BATCHES = (1, 2, 3)
_TC_TOTAL = len(_TC_BATCHES) * _CHUNKS


def _copy_body(lg_ref, v_ref, a_ref, av_ref, o_in_ref, out_ref,
               buf_ref, rsem, wsem):
    es = {b: _argmax3(lg_ref, b) for b in _TC_BATCHES}

    def chunk(t):
        i, j = divmod(t, _CHUNKS)
        return _TC_BATCHES[i], pl.ds(j * _CH, _CH), t % _NBUF

    def read_start(t):
        b, rows, slot = chunk(t)
        for e, src in ((0, v_ref), (1, a_ref), (2, av_ref)):
            @pl.when(es[b] == e)
            def _(src=src):
                pltpu.make_async_copy(
                    src.at[b, rows, :], buf_ref.at[slot], rsem.at[slot]
                ).start()

    def read_wait(t):
        b, rows, slot = chunk(t)
        pltpu.make_async_copy(
            v_ref.at[b, rows, :], buf_ref.at[slot], rsem.at[slot]
        ).wait()

    def write_copy(t):
        b, rows, slot = chunk(t)
        return pltpu.make_async_copy(
            buf_ref.at[slot], out_ref.at[b, rows, :], wsem.at[slot]
        )

    for t in range(_TC_TOTAL + _LAG):
        if t < _TC_TOTAL:
            if t >= _NBUF:
                write_copy(t - _NBUF).wait()
            read_start(t)
        if t >= _LAG:
            read_wait(t - _LAG)
            write_copy(t - _LAG).start()
    for t in range(_TC_TOTAL - _NBUF, _TC_TOTAL):
        write_copy(t).wait()


_SC_NBUF = 4
_SC_CH = 256


def _sc_copy_b0_body(lg_ref, v_ref, a_ref, av_ref, o_ref,
                     lg_smem, buf_ref, lsem, rsem, wsem):
    core = jax.lax.axis_index("core")
    pltpu.make_async_copy(lg_ref, lg_smem, lsem).start()
    pltpu.make_async_copy(lg_ref, lg_smem, lsem).wait()

    l0, l1, l2 = lg_smem[0], lg_smem[1], lg_smem[2]
    e = jnp.where(l2 > jnp.maximum(l0, l1), 2, jnp.where(l1 > l0, 1, 0))

    def rows(t):
        return pl.ds((core * _SC_NBUF + t) * _SC_CH, _SC_CH)

    for t in range(_SC_NBUF):
        for k, src in ((0, v_ref), (1, a_ref), (2, av_ref)):
            @pl.when(e == k)
            def _(src=src, t=t):
                pltpu.make_async_copy(
                    src.at[0, rows(t), :], buf_ref.at[t], rsem.at[t]
                ).start()
    for t in range(_SC_NBUF):
        pltpu.make_async_copy(
            v_ref.at[0, rows(t), :], buf_ref.at[t], rsem.at[t]
        ).wait()
        pltpu.make_async_copy(
            buf_ref.at[t], o_ref.at[0, rows(t), :], wsem.at[t]
        ).start()
    for t in range(_SC_NBUF):
        pltpu.make_async_copy(
            buf_ref.at[t], o_ref.at[0, rows(t), :], wsem.at[t]
        ).wait()


def _sc_copy_b0(logits_b0, v, a, av):
    lg16 = jnp.zeros((16,), jnp.float32).at[:_E].set(logits_b0.reshape(-1))
    f = pl.kernel(
        _sc_copy_b0_body,
        out_type=jax.ShapeDtypeStruct((_B, _S, _D), jnp.float32),
        mesh=plsc.ScalarSubcoreMesh(axis_name="core", num_cores=2),
        scratch_types=[
            pltpu.SMEM((16,), jnp.float32),
            pltpu.VMEM_SHARED((_SC_NBUF, _SC_CH, _D), jnp.float32),
            pltpu.SemaphoreType.DMA,
            pltpu.SemaphoreType.DMA((_SC_NBUF,)),
            pltpu.SemaphoreType.DMA((_SC_NBUF,)),
        ],
    )
    return f(lg16, v, a, av)


def kernel(v0, a0, v, a, av, W, b):
    bias = b.reshape(1, _E)
    lg0 = _stats(v0, a0, W, bias, (0,))
    lg123 = _stats(v0, a0, W, bias, (1, 2, 3))
    logits = jnp.concatenate([lg0, lg123], axis=0)

    partial = _sc_copy_b0(lg0, v, a, av)

    combined = pl.pallas_call(
        _copy_body,
        grid_spec=pltpu.PrefetchScalarGridSpec(
            num_scalar_prefetch=1,
            grid=(1,),
            in_specs=[
                pl.BlockSpec(memory_space=pl.ANY),
                pl.BlockSpec(memory_space=pl.ANY),
                pl.BlockSpec(memory_space=pl.ANY),
                pl.BlockSpec(memory_space=pl.ANY),
            ],
            out_specs=pl.BlockSpec(memory_space=pl.ANY),
            scratch_shapes=[
                pltpu.VMEM((_NBUF, _CH, _D), jnp.float32),
                pltpu.SemaphoreType.DMA((_NBUF,)),
                pltpu.SemaphoreType.DMA((_NBUF,)),
            ],
        ),
        out_shape=jax.ShapeDtypeStruct((_B, _S, _D), jnp.float32),
        input_output_aliases={4: 0},
    )(logits.reshape(_B * _E), v, a, av, partial)

    return combined, logits


# speedup vs baseline: 1.3860x; 1.3642x over previous
import jax
import jax.numpy as jnp
from jax.experimental import pallas as pl
from jax.experimental.pallas import tpu as pltpu

_B, _S, _D, _E = 4, 2048, 1024, 3
_S_BLK = 1024
_S_BLKS = _S // _S_BLK

_CH = 256
_CHUNKS = _S // _CH
_TOTAL = _B * _CHUNKS
_NBUF = 8
_LAG = 4


_SCH = 256
_SCHUNKS = _S // _SCH
_STOTAL = _B * _SCHUNKS
_SNBUF = 6


def _stats_body(v0_ref, a0_ref, w_ref, bias_ref, logits_ref,
                vbuf_ref, abuf_ref, acc_ref, vsem, asem):
    def read_start(t):
        b, j = divmod(t, _SCHUNKS)
        rows = pl.ds(j * _SCH, _SCH)
        slot = t % _SNBUF
        pltpu.make_async_copy(
            v0_ref.at[b, rows, :], vbuf_ref.at[slot], vsem.at[slot]
        ).start()
        pltpu.make_async_copy(
            a0_ref.at[b, rows, :], abuf_ref.at[slot], asem.at[slot]
        ).start()

    def read_wait(t):
        b, j = divmod(t, _SCHUNKS)
        rows = pl.ds(j * _SCH, _SCH)
        slot = t % _SNBUF
        pltpu.make_async_copy(
            v0_ref.at[b, rows, :], vbuf_ref.at[slot], vsem.at[slot]
        ).wait()
        pltpu.make_async_copy(
            a0_ref.at[b, rows, :], abuf_ref.at[slot], asem.at[slot]
        ).wait()

    for t in range(min(_SNBUF, _STOTAL)):
        read_start(t)

    def chunk_sums(buf, slot):
        x0 = buf[slot, 0:8, :]
        s, q = x0, x0 * x0
        for k in range(1, _SCH // 8):
            x = buf[slot, 8 * k:8 * (k + 1), :]
            s = s + x
            q = q + x * x
        return s, q

    for t in range(_STOTAL):
        b, j = divmod(t, _SCHUNKS)
        slot = t % _SNBUF
        read_wait(t)
        sv, qv = chunk_sums(vbuf_ref, slot)
        sa, qa = chunk_sums(abuf_ref, slot)
        if j == 0:
            acc_ref[0:8, :] = sv
            acc_ref[8:16, :] = qv
            acc_ref[16:24, :] = sa
            acc_ref[24:32, :] = qa
        else:
            acc_ref[0:8, :] += sv
            acc_ref[8:16, :] += qv
            acc_ref[16:24, :] += sa
            acc_ref[24:32, :] += qa
        if t + _SNBUF < _STOTAL:
            read_start(t + _SNBUF)
        if j == _SCHUNKS - 1:
            inv_s = 1.0 / _S
            inv_n1 = 1.0 / (_S - 1)
            s_v = jnp.sum(acc_ref[0:8, :], axis=0, keepdims=True)
            q_v = jnp.sum(acc_ref[8:16, :], axis=0, keepdims=True)
            s_a = jnp.sum(acc_ref[16:24, :], axis=0, keepdims=True)
            q_a = jnp.sum(acc_ref[24:32, :], axis=0, keepdims=True)
            mean_v = s_v * inv_s
            var_v = (q_v - _S * mean_v * mean_v) * inv_n1
            mean_a = s_a * inv_s
            var_a = (q_a - _S * mean_a * mean_a) * inv_n1
            feats = jnp.concatenate(
                [mean_v, jnp.sqrt(var_v), mean_a, jnp.sqrt(var_a)], axis=1
            )
            w_bf = w_ref[...].astype(jnp.bfloat16).astype(jnp.float32)
            f_bf = feats.astype(jnp.bfloat16).astype(jnp.float32)
            logits = jnp.sum(w_bf * f_bf, axis=1) + bias_ref[0]
            logits_ref[b, :] = logits


def _argmax3(lg_ref, b):
    l0 = lg_ref[3 * b]
    l1 = lg_ref[3 * b + 1]
    l2 = lg_ref[3 * b + 2]
    i01 = jnp.where(l1 > l0, 1, 0)
    m01 = jnp.maximum(l0, l1)
    return jnp.where(l2 > m01, 2, i01)


def _copy_body(lg_ref, v_ref, a_ref, av_ref, out_ref, buf_ref, rsem, wsem):
    es = [_argmax3(lg_ref, b) for b in range(_B)]

    def chunk(t):
        b, j = divmod(t, _CHUNKS)
        return b, pl.ds(j * _CH, _CH), t % _NBUF

    def read_start(t):
        b, rows, slot = chunk(t)
        for e, src in ((0, v_ref), (1, a_ref), (2, av_ref)):
            @pl.when(es[b] == e)
            def _(src=src):
                pltpu.make_async_copy(
                    src.at[b, rows, :], buf_ref.at[slot], rsem.at[slot]
                ).start()

    def read_wait(t):
        b, rows, slot = chunk(t)
        pltpu.make_async_copy(
            v_ref.at[b, rows, :], buf_ref.at[slot], rsem.at[slot]
        ).wait()

    def write_copy(t):
        b, rows, slot = chunk(t)
        return pltpu.make_async_copy(
            buf_ref.at[slot], out_ref.at[b, rows, :], wsem.at[slot]
        )

    for t in range(_TOTAL + _LAG):
        if t < _TOTAL:
            if t >= _NBUF:
                write_copy(t - _NBUF).wait()
            read_start(t)
        if t >= _LAG:
            read_wait(t - _LAG)
            write_copy(t - _LAG).start()
    for t in range(_TOTAL - _NBUF, _TOTAL):
        write_copy(t).wait()


def kernel(v0, a0, v, a, av, W, b):
    logits = pl.pallas_call(
        _stats_body,
        grid=(1,),
        in_specs=[
            pl.BlockSpec(memory_space=pl.ANY),
            pl.BlockSpec(memory_space=pl.ANY),
            pl.BlockSpec((_E, 4 * _D), lambda i: (0, 0)),
            pl.BlockSpec((1, _E), lambda i: (0, 0)),
        ],
        out_specs=pl.BlockSpec((_B, _E), lambda i: (0, 0)),
        out_shape=jax.ShapeDtypeStruct((_B, _E), jnp.float32),
        scratch_shapes=[
            pltpu.VMEM((_SNBUF, _SCH, _D), jnp.float32),
            pltpu.VMEM((_SNBUF, _SCH, _D), jnp.float32),
            pltpu.VMEM((32, _D), jnp.float32),
            pltpu.SemaphoreType.DMA((_SNBUF,)),
            pltpu.SemaphoreType.DMA((_SNBUF,)),
        ],
    )(v0, a0, W, b.reshape(1, _E))

    combined = pl.pallas_call(
        _copy_body,
        grid_spec=pltpu.PrefetchScalarGridSpec(
            num_scalar_prefetch=1,
            grid=(1,),
            in_specs=[
                pl.BlockSpec(memory_space=pl.ANY),
                pl.BlockSpec(memory_space=pl.ANY),
                pl.BlockSpec(memory_space=pl.ANY),
            ],
            out_specs=pl.BlockSpec(memory_space=pl.ANY),
            scratch_shapes=[
                pltpu.VMEM((_NBUF, _CH, _D), jnp.float32),
                pltpu.SemaphoreType.DMA((_NBUF,)),
                pltpu.SemaphoreType.DMA((_NBUF,)),
            ],
        ),
        out_shape=jax.ShapeDtypeStruct((_B, _S, _D), jnp.float32),
    )(logits.reshape(_B * _E), v, a, av)

    return combined, logits


# speedup vs baseline: 1.4136x; 1.0199x over previous
import jax
import jax.numpy as jnp
from jax.experimental import pallas as pl
from jax.experimental.pallas import tpu as pltpu

_B, _S, _D, _E = 4, 2048, 1024, 3
_S_BLK = 1024
_S_BLKS = _S // _S_BLK

_CH = 512
_CHUNKS = _S // _CH
_TOTAL = _B * _CHUNKS
_NBUF = 6
_LAG = 3


_SCH = 512
_SCHUNKS = _S // _SCH
_STOTAL = _B * _SCHUNKS
_SNBUF = 6


def _stats_body(v0_ref, a0_ref, w_ref, bias_ref, logits_ref,
                vbuf_ref, abuf_ref, acc_ref, vsem, asem):
    def read_start(t):
        b, j = divmod(t, _SCHUNKS)
        rows = pl.ds(j * _SCH, _SCH)
        slot = t % _SNBUF
        pltpu.make_async_copy(
            v0_ref.at[b, rows, :], vbuf_ref.at[slot], vsem.at[slot]
        ).start()
        pltpu.make_async_copy(
            a0_ref.at[b, rows, :], abuf_ref.at[slot], asem.at[slot]
        ).start()

    def read_wait(t):
        b, j = divmod(t, _SCHUNKS)
        rows = pl.ds(j * _SCH, _SCH)
        slot = t % _SNBUF
        pltpu.make_async_copy(
            v0_ref.at[b, rows, :], vbuf_ref.at[slot], vsem.at[slot]
        ).wait()
        pltpu.make_async_copy(
            a0_ref.at[b, rows, :], abuf_ref.at[slot], asem.at[slot]
        ).wait()

    for t in range(min(_SNBUF, _STOTAL)):
        read_start(t)

    def chunk_sums(buf, slot):
        x0 = buf[slot, 0:8, :]
        s, q = x0, x0 * x0
        for k in range(1, _SCH // 8):
            x = buf[slot, 8 * k:8 * (k + 1), :]
            s = s + x
            q = q + x * x
        return s, q

    for t in range(_STOTAL):
        b, j = divmod(t, _SCHUNKS)
        slot = t % _SNBUF
        read_wait(t)
        sv, qv = chunk_sums(vbuf_ref, slot)
        sa, qa = chunk_sums(abuf_ref, slot)
        if j == 0:
            acc_ref[0:8, :] = sv
            acc_ref[8:16, :] = qv
            acc_ref[16:24, :] = sa
            acc_ref[24:32, :] = qa
        else:
            acc_ref[0:8, :] += sv
            acc_ref[8:16, :] += qv
            acc_ref[16:24, :] += sa
            acc_ref[24:32, :] += qa
        if t + _SNBUF < _STOTAL:
            read_start(t + _SNBUF)
        if j == _SCHUNKS - 1:
            inv_s = 1.0 / _S
            inv_n1 = 1.0 / (_S - 1)
            s_v = jnp.sum(acc_ref[0:8, :], axis=0, keepdims=True)
            q_v = jnp.sum(acc_ref[8:16, :], axis=0, keepdims=True)
            s_a = jnp.sum(acc_ref[16:24, :], axis=0, keepdims=True)
            q_a = jnp.sum(acc_ref[24:32, :], axis=0, keepdims=True)
            mean_v = s_v * inv_s
            var_v = (q_v - _S * mean_v * mean_v) * inv_n1
            mean_a = s_a * inv_s
            var_a = (q_a - _S * mean_a * mean_a) * inv_n1
            feats = jnp.concatenate(
                [mean_v, jnp.sqrt(var_v), mean_a, jnp.sqrt(var_a)], axis=1
            )
            w_bf = w_ref[...].astype(jnp.bfloat16).astype(jnp.float32)
            f_bf = feats.astype(jnp.bfloat16).astype(jnp.float32)
            logits = jnp.sum(w_bf * f_bf, axis=1) + bias_ref[0]
            logits_ref[b, :] = logits


def _argmax3(lg_ref, b):
    l0 = lg_ref[3 * b]
    l1 = lg_ref[3 * b + 1]
    l2 = lg_ref[3 * b + 2]
    i01 = jnp.where(l1 > l0, 1, 0)
    m01 = jnp.maximum(l0, l1)
    return jnp.where(l2 > m01, 2, i01)


def _copy_body(lg_ref, v_ref, a_ref, av_ref, out_ref, buf_ref, rsem, wsem):
    es = [_argmax3(lg_ref, b) for b in range(_B)]

    def chunk(t):
        b, j = divmod(t, _CHUNKS)
        return b, pl.ds(j * _CH, _CH), t % _NBUF

    def read_start(t):
        b, rows, slot = chunk(t)
        for e, src in ((0, v_ref), (1, a_ref), (2, av_ref)):
            @pl.when(es[b] == e)
            def _(src=src):
                pltpu.make_async_copy(
                    src.at[b, rows, :], buf_ref.at[slot], rsem.at[slot]
                ).start()

    def read_wait(t):
        b, rows, slot = chunk(t)
        pltpu.make_async_copy(
            v_ref.at[b, rows, :], buf_ref.at[slot], rsem.at[slot]
        ).wait()

    def write_copy(t):
        b, rows, slot = chunk(t)
        return pltpu.make_async_copy(
            buf_ref.at[slot], out_ref.at[b, rows, :], wsem.at[slot]
        )

    for t in range(_TOTAL + _LAG):
        if t < _TOTAL:
            if t >= _NBUF:
                write_copy(t - _NBUF).wait()
            read_start(t)
        if t >= _LAG:
            read_wait(t - _LAG)
            write_copy(t - _LAG).start()
    for t in range(_TOTAL - _NBUF, _TOTAL):
        write_copy(t).wait()


def kernel(v0, a0, v, a, av, W, b):
    logits = pl.pallas_call(
        _stats_body,
        grid=(1,),
        in_specs=[
            pl.BlockSpec(memory_space=pl.ANY),
            pl.BlockSpec(memory_space=pl.ANY),
            pl.BlockSpec((_E, 4 * _D), lambda i: (0, 0)),
            pl.BlockSpec((1, _E), lambda i: (0, 0)),
        ],
        out_specs=pl.BlockSpec((_B, _E), lambda i: (0, 0)),
        out_shape=jax.ShapeDtypeStruct((_B, _E), jnp.float32),
        scratch_shapes=[
            pltpu.VMEM((_SNBUF, _SCH, _D), jnp.float32),
            pltpu.VMEM((_SNBUF, _SCH, _D), jnp.float32),
            pltpu.VMEM((32, _D), jnp.float32),
            pltpu.SemaphoreType.DMA((_SNBUF,)),
            pltpu.SemaphoreType.DMA((_SNBUF,)),
        ],
    )(v0, a0, W, b.reshape(1, _E))

    combined = pl.pallas_call(
        _copy_body,
        grid_spec=pltpu.PrefetchScalarGridSpec(
            num_scalar_prefetch=1,
            grid=(1,),
            in_specs=[
                pl.BlockSpec(memory_space=pl.ANY),
                pl.BlockSpec(memory_space=pl.ANY),
                pl.BlockSpec(memory_space=pl.ANY),
            ],
            out_specs=pl.BlockSpec(memory_space=pl.ANY),
            scratch_shapes=[
                pltpu.VMEM((_NBUF, _CH, _D), jnp.float32),
                pltpu.SemaphoreType.DMA((_NBUF,)),
                pltpu.SemaphoreType.DMA((_NBUF,)),
            ],
        ),
        out_shape=jax.ShapeDtypeStruct((_B, _S, _D), jnp.float32),
    )(logits.reshape(_B * _E), v, a, av)

    return combined, logits
